# baseline (speedup 1.0000x reference)
import jax
import jax.numpy as jnp
from jax import lax
from jax.experimental import pallas as pl
from jax.experimental.pallas import tpu as pltpu

B = 128
N = 4096
TOPK = 32
IOU_CUTOFF = 0.5
EPS = 1e-6
ALPHA = 0.25
GAMMA = 2.0


def _body(s_ref, e_ref, sc_ref, gt_ref, out_ref, cur_ref, mask_ref):
    s = s_ref[:, :]
    e = e_ref[:, :]
    g0 = gt_ref[0:1, :]
    g1 = gt_ref[1:2, :]

    inter = jnp.clip(jnp.minimum(e, g1) - jnp.maximum(s, g0), 0.0)
    la = e - s
    lb = g1 - g0
    union = la + lb - inter
    enclose = jnp.maximum(e, g1) - jnp.minimum(s, g0)
    iou = inter / (union + EPS)
    giou = iou - (enclose - union) / (enclose + EPS)

    rowid = lax.broadcasted_iota(jnp.int32, (N, B), 0)

    cur_ref[:, :] = giou
    mask_ref[:, :] = jnp.zeros((N, B), jnp.float32)

    def step(_, dummy):
        cur = cur_ref[:, :]
        m = jnp.max(cur, axis=0, keepdims=True)
        idx = jnp.min(jnp.where(cur == m, rowid, N), axis=0, keepdims=True)
        hit = rowid == idx
        cur_ref[:, :] = jnp.where(hit, -3.0, cur)
        mask_ref[:, :] = jnp.where(hit, 1.0, mask_ref[:, :])
        return dummy

    lax.fori_loop(0, TOPK, step, 0)
    mask = mask_ref[:, :] > 0.5

    t = jnp.where(mask, 1.0, jnp.where(giou < IOU_CUTOFF, 0.0, giou))

    l = sc_ref[:, :]
    q = jnp.exp(-jnp.abs(l))
    ce = jnp.maximum(l, 0.0) - l * t + jnp.log1p(q)
    p = jax.nn.sigmoid(l)
    p_t = p * t + (1.0 - p) * (1.0 - t)
    alpha_t = ALPHA * t + (1.0 - ALPHA) * (1.0 - t)
    focal_sum = jnp.sum(alpha_t * ce * (1.0 - p_t) ** 2)
    val_iou_loss = focal_sum / (B * N)

    l1 = jnp.abs(s - g0) + jnp.abs(e - g1)
    l1_sum = jnp.sum(jnp.where(mask, l1, 0.0))
    val_l1_loss = l1_sum / (B * TOPK * 2)

    out_ref[0, 0] = val_iou_loss + val_l1_loss


def kernel(proposal, score, gt):
    s = proposal[:, :, 0].T
    e = proposal[:, :, 1].T
    out = pl.pallas_call(
        _body,
        out_shape=jax.ShapeDtypeStruct((1, 1), jnp.float32),
        out_specs=pl.BlockSpec(memory_space=pltpu.SMEM),
        scratch_shapes=[
            pltpu.VMEM((N, B), jnp.float32),
            pltpu.VMEM((N, B), jnp.float32),
        ],
    )(s, e, score.T, gt.T)
    return out[0, 0]


# speedup vs baseline: 1.5458x; 1.5458x over previous
import jax
import jax.numpy as jnp
from jax import lax
from jax.experimental import pallas as pl
from jax.experimental.pallas import tpu as pltpu

B = 128
N = 4096
TOPK = 32
IOU_CUTOFF = 0.5
EPS = 1e-6
ALPHA = 0.25
GAMMA = 2.0
TIE = 2.0 ** -28


def _body(s_ref, e_ref, sc_ref, gt_ref, out_ref, cur_ref, mask_ref):
    s = s_ref[:, :]
    e = e_ref[:, :]
    g0 = gt_ref[:, 0:1]
    g1 = gt_ref[:, 1:2]

    inter = jnp.clip(jnp.minimum(e, g1) - jnp.maximum(s, g0), 0.0)
    la = e - s
    lb = g1 - g0
    union = la + lb - inter
    enclose = jnp.maximum(e, g1) - jnp.minimum(s, g0)
    iou = inter / (union + EPS)
    giou = iou - (enclose - union) / (enclose + EPS)

    colid = lax.broadcasted_iota(jnp.int32, (B, N), 1)
    key = giou - colid.astype(jnp.float32) * TIE

    cur_ref[:, :] = key
    mask_ref[:, :] = jnp.zeros((B, N), jnp.float32)

    def step(_, dummy):
        cur = cur_ref[:, :]
        m = jnp.max(cur, axis=1, keepdims=True)
        hit = cur == m
        cur_ref[:, :] = jnp.where(hit, -3.0, cur)
        mask_ref[:, :] = jnp.where(hit, 1.0, mask_ref[:, :])
        return dummy

    lax.fori_loop(0, TOPK, step, 0)
    mask = mask_ref[:, :] > 0.5

    t = jnp.where(mask, 1.0, jnp.where(giou < IOU_CUTOFF, 0.0, giou))

    l = sc_ref[:, :]
    q = jnp.exp(-jnp.abs(l))
    ce = jnp.maximum(l, 0.0) - l * t + jnp.log1p(q)
    p = jax.nn.sigmoid(l)
    p_t = p * t + (1.0 - p) * (1.0 - t)
    alpha_t = ALPHA * t + (1.0 - ALPHA) * (1.0 - t)
    focal_sum = jnp.sum(alpha_t * ce * (1.0 - p_t) ** 2)
    val_iou_loss = focal_sum / (B * N)

    l1 = jnp.abs(s - g0) + jnp.abs(e - g1)
    l1_sum = jnp.sum(jnp.where(mask, l1, 0.0))
    val_l1_loss = l1_sum / (B * TOPK * 2)

    out_ref[0, 0] = val_iou_loss + val_l1_loss


def kernel(proposal, score, gt):
    s = proposal[:, :, 0]
    e = proposal[:, :, 1]
    out = pl.pallas_call(
        _body,
        out_shape=jax.ShapeDtypeStruct((1, 1), jnp.float32),
        out_specs=pl.BlockSpec(memory_space=pltpu.SMEM),
        scratch_shapes=[
            pltpu.VMEM((B, N), jnp.float32),
            pltpu.VMEM((B, N), jnp.float32),
        ],
    )(s, e, score, gt)
    return out[0, 0]


# speedup vs baseline: 1.9892x; 1.2868x over previous
import jax
import jax.numpy as jnp
from jax import lax
from jax.experimental import pallas as pl
from jax.experimental.pallas import tpu as pltpu

B = 128
N = 4096
TOPK = 32
IOU_CUTOFF = 0.5
EPS = 1e-6
ALPHA = 0.25
GAMMA = 2.0
TIE = 2.0 ** -28


def _body(s_ref, e_ref, sc_ref, gt_ref, out_ref, cur_ref):
    s = s_ref[:, :]
    e = e_ref[:, :]
    g0 = gt_ref[:, 0:1]
    g1 = gt_ref[:, 1:2]

    inter = jnp.clip(jnp.minimum(e, g1) - jnp.maximum(s, g0), 0.0)
    la = e - s
    lb = g1 - g0
    union = la + lb - inter
    enclose = jnp.maximum(e, g1) - jnp.minimum(s, g0)
    iou = inter / (union + EPS)
    giou = iou - (enclose - union) / (enclose + EPS)

    colid = lax.broadcasted_iota(jnp.int32, (B, N), 1)
    key = giou - colid.astype(jnp.float32) * TIE

    cur_ref[:, :] = key

    def step(_, dummy):
        cur = cur_ref[:, :]
        m = jnp.max(cur, axis=1, keepdims=True)
        cur_ref[:, :] = jnp.where(cur == m, -3.0, cur)
        return dummy

    lax.fori_loop(0, TOPK, step, 0)
    mask = cur_ref[:, :] == -3.0

    t = jnp.where(mask, 1.0, jnp.where(giou < IOU_CUTOFF, 0.0, giou))

    l = sc_ref[:, :]
    q = jnp.exp(-jnp.abs(l))
    ce = jnp.maximum(l, 0.0) - l * t + jnp.log1p(q)
    r = 1.0 / (1.0 + q)
    p = jnp.where(l >= 0.0, r, q * r)
    p_t = p * t + (1.0 - p) * (1.0 - t)
    alpha_t = ALPHA * t + (1.0 - ALPHA) * (1.0 - t)
    focal_sum = jnp.sum(alpha_t * ce * (1.0 - p_t) ** 2)
    val_iou_loss = focal_sum / (B * N)

    l1 = jnp.abs(s - g0) + jnp.abs(e - g1)
    l1_sum = jnp.sum(jnp.where(mask, l1, 0.0))
    val_l1_loss = l1_sum / (B * TOPK * 2)

    out_ref[0, 0] = val_iou_loss + val_l1_loss


def kernel(proposal, score, gt):
    s = proposal[:, :, 0]
    e = proposal[:, :, 1]
    out = pl.pallas_call(
        _body,
        out_shape=jax.ShapeDtypeStruct((1, 1), jnp.float32),
        out_specs=pl.BlockSpec(memory_space=pltpu.SMEM),
        scratch_shapes=[
            pltpu.VMEM((B, N), jnp.float32),
        ],
    )(s, e, score, gt)
    return out[0, 0]
